# Initial kernel scaffold; baseline (speedup 1.0000x reference)
import jax
import jax.numpy as jnp
from jax.experimental import pallas as pl


def kernel(input_ids, data_tensor, embed_table, out_w, out_b):
    raise NotImplementedError("write your pallas kernel here")


# speedup vs baseline: 4.3518x; 4.3518x over previous
import functools

import jax
import jax.numpy as jnp
from jax import lax
from jax.experimental import pallas as pl
from jax.experimental.pallas import tpu as pltpu
from jax.experimental.pallas import tpu_sc as plsc

NW = 32
LANES = 16


def _make_sc_lookup(n_ids: int):
    per_w = n_ids // NW
    chunk = 6400
    assert per_w % chunk == 0
    nchunk = per_w // chunk
    groups = chunk // LANES

    mesh = plsc.VectorSubcoreMesh(core_axis_name="c", subcore_axis_name="s")

    @functools.partial(
        pl.kernel,
        out_type=jax.ShapeDtypeStruct((n_ids * 6,), jnp.float32),
        mesh=mesh,
        scratch_types=[
            pltpu.VMEM((chunk,), jnp.int32),
            pltpu.VMEM((chunk * 6,), jnp.float32),
            pltpu.VMEM((64,), jnp.float32),
            pltpu.VMEM((16, 4), jnp.float32),
            pltpu.VMEM((4, 6), jnp.float32),
            pltpu.VMEM((16,), jnp.float32),
        ],
        compiler_params=pltpu.CompilerParams(needs_layout_passes=False),
    )
    def sc_lookup(ids_hbm, etab_hbm, w_hbm, b_hbm, out_hbm,
                  ids_v, out_v, fused_v, etab_v, w_v, b_v):
        wid = lax.axis_index("s") * 2 + lax.axis_index("c")
        base = wid * per_w

        pltpu.sync_copy(etab_hbm, etab_v.at[pl.ds(0, 10)])
        pltpu.sync_copy(w_hbm, w_v)
        pltpu.sync_copy(b_hbm, b_v.at[pl.ds(0, 6)])

        m = lax.iota(jnp.int32, 16)
        for j in range(4):
            p = m + 16 * j
            v = p // 6
            e = p - v * 6
            acc = plsc.load_gather(b_v, [e])
            for d in range(4):
                dd = jnp.full((16,), d, jnp.int32)
                acc = acc + (plsc.load_gather(etab_v, [v, dd])
                             * plsc.load_gather(w_v, [dd, e]))
            fused_v[pl.ds(16 * j, 16)] = acc

        def chunk_body(c, _):
            off = base + c * chunk
            pltpu.sync_copy(ids_hbm.at[pl.ds(off, chunk)], ids_v)

            def pre(g, _):
                sl = pl.ds(g * 16, 16)
                ids_v[sl] = ids_v[sl] * 6
                return 0

            lax.fori_loop(0, groups, pre, 0, unroll=4)

            def grp(g, _):
                gi = g * 16
                ob = g * 96
                for j in range(6):
                    p = m + 16 * j
                    lsel = p // 6
                    esel = p - lsel * 6
                    idx = plsc.load_gather(ids_v, [gi + lsel]) + esel
                    out_v[pl.ds(ob + j * 16, 16)] = plsc.load_gather(
                        fused_v, [idx])
                return 0

            lax.fori_loop(0, groups, grp, 0)
            pltpu.sync_copy(out_v, out_hbm.at[pl.ds(off * 6, chunk * 6)])
            return 0

        lax.fori_loop(0, nchunk, chunk_body, 0)

    return sc_lookup


def kernel(input_ids, data_tensor, embed_table, out_w, out_b):
    b, l = input_ids.shape
    ids = input_ids.reshape(-1).astype(jnp.int32)
    out_flat = _make_sc_lookup(ids.shape[0])(ids, embed_table, out_w, out_b)
    return out_flat.reshape(b, l, 6)


# speedup vs baseline: 48.7099x; 11.1930x over previous
import functools

import jax
import jax.numpy as jnp
from jax import lax
from jax.experimental import pallas as pl
from jax.experimental.pallas import tpu as pltpu
from jax.experimental.pallas import tpu_sc as plsc

NW = 32
LANES = 16


def _make_sc_lookup(n_b: int, n_l: int):
    nbt = n_b // 128
    nlt = n_l // 8
    bcw = nbt // NW

    mesh = plsc.VectorSubcoreMesh(core_axis_name="c", subcore_axis_name="s")

    @functools.partial(
        pl.kernel,
        out_type=jax.ShapeDtypeStruct((6, nlt, nbt, 8, 128), jnp.float32),
        mesh=mesh,
        scratch_types=[
            pltpu.VMEM((bcw, 8, 128), jnp.int32),
            pltpu.VMEM((6, bcw, 8, 128), jnp.float32),
            pltpu.VMEM((64,), jnp.float32),
            pltpu.VMEM((16, 4), jnp.float32),
            pltpu.VMEM((4, 6), jnp.float32),
            pltpu.VMEM((16,), jnp.float32),
            pltpu.SemaphoreType.DMA,
        ],
        compiler_params=pltpu.CompilerParams(needs_layout_passes=False),
    )
    def sc_lookup(ids_hbm, etab_hbm, w_hbm, b_hbm, out_hbm,
                  ids_v, out_v, fused_v, etab_v, w_v, b_v, sem):
        wid = lax.axis_index("s") * 2 + lax.axis_index("c")
        bc0 = wid * bcw

        pltpu.sync_copy(etab_hbm, etab_v.at[pl.ds(0, 10)])
        pltpu.sync_copy(w_hbm, w_v)
        pltpu.sync_copy(b_hbm, b_v.at[pl.ds(0, 6)])

        m = lax.iota(jnp.int32, 16)
        for j in range(4):
            p = m + 16 * j
            v = p // 6
            e = p - v * 6
            acc = plsc.load_gather(b_v, [e])
            for d in range(4):
                dd = jnp.full((16,), d, jnp.int32)
                acc = acc + (plsc.load_gather(etab_v, [v, dd])
                             * plsc.load_gather(w_v, [dd, e]))
            fused_v[pl.ds(16 * j, 16)] = acc

        def lt_body(lt, _):
            pltpu.sync_copy(ids_hbm.at[lt, pl.ds(bc0, bcw)], ids_v)

            def vec_body(i, _):
                bcl = i // 64
                ls = (i // 8) - bcl * 8
                k = i - (i // 8) * 8
                id6 = ids_v[bcl, ls, pl.ds(k * 16, 16)] * 6
                for e in range(6):
                    out_v[e, bcl, ls, pl.ds(k * 16, 16)] = plsc.load_gather(
                        fused_v, [id6 + e])
                return 0

            lax.fori_loop(0, bcw * 64, vec_body, 0)

            cps = [pltpu.async_copy(out_v.at[e],
                                    out_hbm.at[e, lt, pl.ds(bc0, bcw)], sem)
                   for e in range(6)]
            for cp in cps:
                cp.wait()
            return 0

        lax.fori_loop(0, nlt, lt_body, 0)

    return sc_lookup


def kernel(input_ids, data_tensor, embed_table, out_w, out_b):
    b, l = input_ids.shape
    nbt, nlt = b // 128, l // 8
    ids4 = (input_ids.astype(jnp.int32)
            .reshape(nbt, 128, nlt, 8)
            .transpose(2, 0, 3, 1))
    t = _make_sc_lookup(b, l)(ids4, embed_table, out_w, out_b)
    return t.transpose(2, 4, 1, 3, 0).reshape(b, l, 6)


# speedup vs baseline: 136.6998x; 2.8064x over previous
import functools

import jax
import jax.numpy as jnp
from jax import lax
from jax.experimental import pallas as pl
from jax.experimental.pallas import tpu as pltpu
from jax.experimental.pallas import tpu_sc as plsc

NW = 32


def _make_sc_lookup(n_b: int, n_l: int):
    nbt = n_b // 128
    nlt = n_l // 8
    bcw = nbt // NW
    blk = bcw * 1024
    nvec = blk // 16

    mesh = plsc.VectorSubcoreMesh(core_axis_name="c", subcore_axis_name="s")

    @functools.partial(
        pl.kernel,
        out_type=jax.ShapeDtypeStruct((6, nlt, nbt, 8, 128), jnp.float32),
        mesh=mesh,
        scratch_types=[
            pltpu.VMEM((bcw, 8, 128), jnp.int32),
            pltpu.VMEM((6, bcw, 8, 128), jnp.float32),
            pltpu.VMEM((64,), jnp.float32),
            pltpu.VMEM((16, 4), jnp.float32),
            pltpu.VMEM((4, 6), jnp.float32),
            pltpu.VMEM((16,), jnp.float32),
            pltpu.SemaphoreType.DMA,
            pltpu.SemaphoreType.DMA,
        ],
        compiler_params=pltpu.CompilerParams(needs_layout_passes=False),
    )
    def sc_lookup(ids_hbm, etab_hbm, w_hbm, b_hbm, out_hbm,
                  ids_v, out_v, tab_v, etab_v, w_v, b_v, sem_in, sem_out):
        wid = lax.axis_index("s") * 2 + lax.axis_index("c")
        bc0 = wid * bcw

        pltpu.sync_copy(etab_hbm, etab_v.at[pl.ds(0, 10)])
        pltpu.sync_copy(w_hbm, w_v)
        pltpu.sync_copy(b_hbm, b_v.at[pl.ds(0, 6)])

        m = lax.iota(jnp.int32, 16)
        for j in range(4):
            p = m + 16 * j
            v = p // 6
            e = p - v * 6
            acc = plsc.load_gather(b_v, [e])
            for d in range(4):
                dd = jnp.full((16,), d, jnp.int32)
                acc = acc + (plsc.load_gather(etab_v, [v, dd])
                             * plsc.load_gather(w_v, [dd, e]))
            tab_v[pl.ds(16 * j, 16)] = acc

        def lt_body(lt, _):
            pltpu.sync_copy(ids_hbm.at[lt, pl.ds(bc0, bcw)], ids_v)

            @plsc.parallel_loop(0, nvec, unroll=4)
            def _(j):
                bcl = j // 64
                ls = (j // 8) - bcl * 8
                k = j - (j // 8) * 8
                sl = pl.ds(k * 16, 16)
                id6 = ids_v[bcl, ls, sl] * 6
                for e in range(6):
                    out_v[e, bcl, ls, sl] = plsc.load_gather(
                        tab_v, [id6 + e])

            cps = [pltpu.async_copy(out_v.at[e],
                                    out_hbm.at[e, lt, pl.ds(bc0, bcw)],
                                    sem_out)
                   for e in range(6)]
            for cp in cps:
                cp.wait()
            return 0

        lax.fori_loop(0, nlt, lt_body, 0)

    return sc_lookup


def kernel(input_ids, data_tensor, embed_table, out_w, out_b):
    b, l = input_ids.shape
    nbt, nlt = b // 128, l // 8
    ids4 = (input_ids.astype(jnp.int32)
            .reshape(nbt, 128, nlt, 8)
            .transpose(2, 0, 3, 1))
    t = _make_sc_lookup(b, l)(ids4, embed_table, out_w, out_b)
    return t.transpose(2, 4, 1, 3, 0).reshape(b, l, 6)


# speedup vs baseline: 223.6443x; 1.6360x over previous
import functools

import jax
import jax.numpy as jnp
from jax import lax
from jax.experimental import pallas as pl
from jax.experimental.pallas import tpu as pltpu
from jax.experimental.pallas import tpu_sc as plsc

NW = 32


def _make_sc_lookup(n_b: int, n_l: int):
    nbt = n_b // 128
    nlt = n_l // 8
    bcw = nbt // NW
    blk = bcw * 1024
    nvec = blk // 16

    mesh = plsc.VectorSubcoreMesh(core_axis_name="c", subcore_axis_name="s")

    @functools.partial(
        pl.kernel,
        out_type=jax.ShapeDtypeStruct((6, nlt, nbt, 8, 128), jnp.float32),
        mesh=mesh,
        scratch_types=[
            pltpu.VMEM((2, bcw, 8, 128), jnp.int32),
            pltpu.VMEM((2, 6, bcw, 8, 128), jnp.float32),
            pltpu.VMEM((64,), jnp.float32),
            pltpu.VMEM((16, 4), jnp.float32),
            pltpu.VMEM((4, 6), jnp.float32),
            pltpu.VMEM((16,), jnp.float32),
            pltpu.SemaphoreType.DMA,
            pltpu.SemaphoreType.DMA,
        ],
        compiler_params=pltpu.CompilerParams(needs_layout_passes=False),
    )
    def sc_lookup(ids_hbm, etab_hbm, w_hbm, b_hbm, out_hbm,
                  ids_v, out_v, tab_v, etab_v, w_v, b_v, sem_in, sem_out):
        wid = lax.axis_index("s") * 2 + lax.axis_index("c")
        bc0 = wid * bcw

        pltpu.sync_copy(etab_hbm, etab_v.at[pl.ds(0, 10)])
        pltpu.sync_copy(w_hbm, w_v)
        pltpu.sync_copy(b_hbm, b_v.at[pl.ds(0, 6)])

        m = lax.iota(jnp.int32, 16)
        for j in range(4):
            p = m + 16 * j
            v = p // 6
            e = p - v * 6
            acc = plsc.load_gather(b_v, [e])
            for d in range(4):
                dd = jnp.full((16,), d, jnp.int32)
                acc = acc + (plsc.load_gather(etab_v, [v, dd])
                             * plsc.load_gather(w_v, [dd, e]))
            tab_v[pl.ds(16 * j, 16)] = acc

        def ids_cp(lt, par):
            return pltpu.make_async_copy(
                ids_hbm.at[lt, pl.ds(bc0, bcw)], ids_v.at[par], sem_in)

        def out_cp(lt, par, e):
            return pltpu.make_async_copy(
                out_v.at[par, e], out_hbm.at[e, lt, pl.ds(bc0, bcw)], sem_out)

        ids_cp(0, 0).start()

        def lt_body(lt, _):
            par = lax.rem(lt, 2)
            ids_cp(lt, par).wait()

            @pl.when(lt + 1 < nlt)
            def _():
                ids_cp(lt + 1, 1 - par).start()

            @pl.when(lt >= 2)
            def _():
                for e in range(6):
                    out_cp(lt - 2, par, e).wait()

            @plsc.parallel_loop(0, nvec, unroll=4)
            def _(j):
                bcl = j // 64
                ls = (j // 8) - bcl * 8
                k = j - (j // 8) * 8
                sl = pl.ds(k * 16, 16)
                id6 = ids_v[par, bcl, ls, sl] * 6
                for e in range(6):
                    out_v[par, e, bcl, ls, sl] = plsc.load_gather(
                        tab_v, [id6 + e])

            for e in range(6):
                out_cp(lt, par, e).start()
            return 0

        lax.fori_loop(0, nlt, lt_body, 0)

        for lt in (nlt - 2, nlt - 1):
            for e in range(6):
                out_cp(lt, lax.rem(lt, 2), e).wait()

    return sc_lookup


def kernel(input_ids, data_tensor, embed_table, out_w, out_b):
    b, l = input_ids.shape
    nbt, nlt = b // 128, l // 8
    ids4 = (input_ids.astype(jnp.int32)
            .reshape(nbt, 128, nlt, 8)
            .transpose(2, 0, 3, 1))
    t = _make_sc_lookup(b, l)(ids4, embed_table, out_w, out_b)
    return t.transpose(2, 4, 1, 3, 0).reshape(b, l, 6)


# speedup vs baseline: 229.4591x; 1.0260x over previous
import functools

import jax
import jax.numpy as jnp
from jax import lax
from jax.experimental import pallas as pl
from jax.experimental.pallas import tpu as pltpu
from jax.experimental.pallas import tpu_sc as plsc

NW = 32


def _make_sc_lookup(n_b: int, n_l: int):
    nbt = n_b // 128
    nlt = n_l // 8
    bcw = nbt // NW
    blk = bcw * 1024
    nvec = blk // 16

    mesh = plsc.VectorSubcoreMesh(core_axis_name="c", subcore_axis_name="s")

    @functools.partial(
        pl.kernel,
        out_type=jax.ShapeDtypeStruct((6, nlt, nbt, 8, 128), jnp.float32),
        mesh=mesh,
        scratch_types=[
            pltpu.VMEM((2, bcw, 8, 128), jnp.int32),
            pltpu.VMEM((2, 6, bcw, 8, 128), jnp.float32),
            pltpu.VMEM((64,), jnp.float32),
            pltpu.VMEM((16, 4), jnp.float32),
            pltpu.VMEM((4, 6), jnp.float32),
            pltpu.VMEM((16,), jnp.float32),
            pltpu.SemaphoreType.DMA,
            pltpu.SemaphoreType.DMA,
        ],
        compiler_params=pltpu.CompilerParams(needs_layout_passes=False),
    )
    def sc_lookup(ids_hbm, etab_hbm, w_hbm, b_hbm, out_hbm,
                  ids_v, out_v, tab_v, etab_v, w_v, b_v, sem_in, sem_out):
        wid = lax.axis_index("s") * 2 + lax.axis_index("c")
        bc0 = wid * bcw

        def ids_cp(lt, par):
            return pltpu.make_async_copy(
                ids_hbm.at[lt, pl.ds(bc0, bcw)], ids_v.at[par], sem_in)

        def out_cp(lt, par, e):
            return pltpu.make_async_copy(
                out_v.at[par, e], out_hbm.at[e, lt, pl.ds(bc0, bcw)], sem_out)

        ids_cp(0, 0).start()

        wcps = [pltpu.make_async_copy(etab_hbm, etab_v.at[pl.ds(0, 10)],
                                      sem_out),
                pltpu.make_async_copy(w_hbm, w_v, sem_out),
                pltpu.make_async_copy(b_hbm, b_v.at[pl.ds(0, 6)], sem_out)]
        for cp in wcps:
            cp.start()
        for cp in wcps:
            cp.wait()

        m = lax.iota(jnp.int32, 16)
        for j in range(4):
            p = m + 16 * j
            v = p // 6
            e = p - v * 6
            acc = plsc.load_gather(b_v, [e])
            for d in range(4):
                dd = jnp.full((16,), d, jnp.int32)
                acc = acc + (plsc.load_gather(etab_v, [v, dd])
                             * plsc.load_gather(w_v, [dd, e]))
            tab_v[pl.ds(16 * j, 16)] = acc

        def lt_body(lt, _):
            par = lax.rem(lt, 2)
            ids_cp(lt, par).wait()

            @pl.when(lt + 1 < nlt)
            def _():
                ids_cp(lt + 1, 1 - par).start()

            @pl.when(lt >= 2)
            def _():
                for e in range(6):
                    out_cp(lt - 2, par, e).wait()

            @plsc.parallel_loop(0, nvec, unroll=4)
            def _(j):
                bcl = j // 64
                ls = (j // 8) - bcl * 8
                k = j - (j // 8) * 8
                sl = pl.ds(k * 16, 16)
                id6 = ids_v[par, bcl, ls, sl] * 6
                for e in range(6):
                    out_v[par, e, bcl, ls, sl] = plsc.load_gather(
                        tab_v, [id6 + e])

            for e in range(6):
                out_cp(lt, par, e).start()
            return 0

        lax.fori_loop(0, nlt, lt_body, 0)

        for lt in (nlt - 2, nlt - 1):
            for e in range(6):
                out_cp(lt, lax.rem(lt, 2), e).wait()

    return sc_lookup


def kernel(input_ids, data_tensor, embed_table, out_w, out_b):
    b, l = input_ids.shape
    nbt, nlt = b // 128, l // 8
    ids4 = (input_ids.astype(jnp.int32)
            .reshape(nbt, 128, nlt, 8)
            .transpose(2, 0, 3, 1))
    t = _make_sc_lookup(b, l)(ids4, embed_table, out_w, out_b)
    return t.transpose(2, 4, 1, 3, 0).reshape(b, l, 6)


# speedup vs baseline: 230.1520x; 1.0030x over previous
import functools

import jax
import jax.numpy as jnp
from jax import lax
from jax.experimental import pallas as pl
from jax.experimental.pallas import tpu as pltpu
from jax.experimental.pallas import tpu_sc as plsc

NW = 32


def _make_sc_lookup(n_b: int, n_l: int):
    nbt = n_b // 128
    nlt = n_l // 8
    bcw = nbt // NW
    nvec = bcw * 64

    mesh = plsc.VectorSubcoreMesh(core_axis_name="c", subcore_axis_name="s")

    @functools.partial(
        pl.kernel,
        out_type=jax.ShapeDtypeStruct((6, nlt, nbt, 8, 128), jnp.float32),
        mesh=mesh,
        scratch_types=[
            pltpu.VMEM((2, bcw, 8, 128), jnp.int32),
            pltpu.VMEM((2, 6, bcw, 8, 128), jnp.float32),
            pltpu.VMEM((64,), jnp.float32),
            pltpu.VMEM((16, 4), jnp.float32),
            pltpu.VMEM((4, 6), jnp.float32),
            pltpu.VMEM((16,), jnp.float32),
            pltpu.SemaphoreType.DMA,
            pltpu.SemaphoreType.DMA,
        ],
        compiler_params=pltpu.CompilerParams(needs_layout_passes=False),
    )
    def sc_lookup(ids_hbm, etab_hbm, w_hbm, b_hbm, out_hbm,
                  ids_v, out_v, tab_v, etab_v, w_v, b_v, sem_in, sem_out):
        wid = lax.axis_index("s") * 2 + lax.axis_index("c")
        bc0 = wid * bcw

        def ids_cp(lt, par):
            return pltpu.make_async_copy(
                ids_hbm.at[lt, pl.ds(bc0, bcw)], ids_v.at[par], sem_in)

        def out_cp(lt, par, e):
            return pltpu.make_async_copy(
                out_v.at[par, e], out_hbm.at[e, lt, pl.ds(bc0, bcw)], sem_out)

        ids_cp(0, 0).start()

        wcps = [pltpu.make_async_copy(etab_hbm, etab_v.at[pl.ds(0, 10)],
                                      sem_out),
                pltpu.make_async_copy(w_hbm, w_v, sem_out),
                pltpu.make_async_copy(b_hbm, b_v.at[pl.ds(0, 6)], sem_out)]
        for cp in wcps:
            cp.start()
        for cp in wcps:
            cp.wait()

        m = lax.iota(jnp.int32, 16)
        for j in range(4):
            p = m + 16 * j
            v = p // 6
            e = p - v * 6
            acc = plsc.load_gather(b_v, [e])
            for d in range(4):
                dd = jnp.full((16,), d, jnp.int32)
                acc = acc + (plsc.load_gather(etab_v, [v, dd])
                             * plsc.load_gather(w_v, [dd, e]))
            tab_v[pl.ds(16 * j, 16)] = acc

        def lt_body(lt, _):
            par = lax.rem(lt, 2)
            ids_cp(lt, par).wait()

            @pl.when(lt + 1 < nlt)
            def _():
                ids_cp(lt + 1, 1 - par).start()

            @pl.when(lt >= 2)
            def _():
                for e in range(6):
                    out_cp(lt - 2, par, e).wait()

            @plsc.parallel_loop(0, nvec, unroll=4)
            def _(j):
                bcl = j // 64
                ls = (j // 8) - bcl * 8
                k = j - (j // 8) * 8
                sl = pl.ds(k * 16, 16)
                id6 = ids_v[par, bcl, ls, sl] * 6
                for e in range(6):
                    out_v[par, e, bcl, ls, sl] = plsc.load_gather(
                        tab_v, [id6 + e])

            for e in range(6):
                out_cp(lt, par, e).start()
            return 0

        lax.fori_loop(0, nlt, lt_body, 0)

        for lt in (nlt - 2, nlt - 1):
            for e in range(6):
                out_cp(lt, lax.rem(lt, 2), e).wait()

    return sc_lookup


def kernel(input_ids, data_tensor, embed_table, out_w, out_b):
    b, l = input_ids.shape
    nbt, nlt = b // 128, l // 8
    ids4 = (input_ids.astype(jnp.int32)
            .reshape(nbt, 128, nlt, 8)
            .transpose(2, 0, 3, 1))
    t = _make_sc_lookup(b, l)(ids4, embed_table, out_w, out_b)
    return t.transpose(2, 4, 1, 3, 0).reshape(b, l, 6)
